# baseline (speedup 1.0000x reference)
import functools

import jax
import jax.numpy as jnp
from jax import lax
from jax.experimental import pallas as pl
from jax.experimental.pallas import tpu as pltpu
from jax.experimental.pallas import tpu_sc as plsc

_NC = 2
_NS = 16
_NW = _NC * _NS
_R = 32


def kernel(x, pe):
    B, S, D = x.shape
    rows_per_w = S // _NW
    n_chunks = rows_per_w // _R
    mesh = plsc.VectorSubcoreMesh(core_axis_name="c", subcore_axis_name="s")

    @functools.partial(
        pl.kernel,
        mesh=mesh,
        out_type=jax.ShapeDtypeStruct((B, S, D), jnp.float32),
        scratch_types=[
            pltpu.VMEM((_R, D), jnp.float32),
            pltpu.VMEM((_R, D), jnp.float32),
        ],
    )
    def k(x_hbm, pe_hbm, out_hbm, pe_v, x_v):
        wid = lax.axis_index("s") * _NC + lax.axis_index("c")
        base = wid * rows_per_w
        for c in range(n_chunks):
            row0 = base + c * _R
            pltpu.sync_copy(pe_hbm.at[pl.ds(row0, _R), :], pe_v)
            for b in range(B):
                pltpu.sync_copy(x_hbm.at[b, pl.ds(row0, _R), :], x_v)

                def add_body(i, carry):
                    r = i >> 3
                    j = (i & 7) * 128
                    for u in range(8):
                        sl = pl.ds(j + u * 16, 16)
                        plsc.addupdate(x_v.at[r, sl], pe_v[r, sl])
                    return carry

                lax.fori_loop(0, _R * 8, add_body, 0)
                pltpu.sync_copy(x_v, out_hbm.at[b, pl.ds(row0, _R), :])

    return k(x, pe)


# speedup vs baseline: 1.2563x; 1.2563x over previous
import functools

import jax
import jax.numpy as jnp
from jax import lax
from jax.experimental import pallas as pl
from jax.experimental.pallas import tpu as pltpu
from jax.experimental.pallas import tpu_sc as plsc

_NC = 2
_NS = 16
_NW = _NC * _NS
_R = 16


def kernel(x, pe):
    B, S, D = x.shape
    rows_per_w = S // _NW
    n_chunks = rows_per_w // _R
    n_items = n_chunks * B
    mesh = plsc.VectorSubcoreMesh(core_axis_name="c", subcore_axis_name="s")

    @functools.partial(
        pl.kernel,
        mesh=mesh,
        out_type=jax.ShapeDtypeStruct((B, S, D), jnp.float32),
        scratch_types=[
            pltpu.VMEM((2, _R, D), jnp.float32),
            pltpu.VMEM((2, _R, D), jnp.float32),
            pltpu.SemaphoreType.DMA,
            pltpu.SemaphoreType.DMA,
            pltpu.SemaphoreType.DMA,
        ],
    )
    def k(x_hbm, pe_hbm, out_hbm, pe_v, x_v, in_sem, pe_sem, out_sem):
        wid = lax.axis_index("s") * _NC + lax.axis_index("c")
        base = wid * rows_per_w
        inh = [None] * n_items
        peh = [None] * n_chunks
        outh = [None] * n_items
        for it in range(n_items + 1):
            if it < n_items:
                c, b = divmod(it, B)
                row0 = base + c * _R
                if b == 0:
                    peh[c] = pltpu.async_copy(
                        pe_hbm.at[pl.ds(row0, _R), :], pe_v.at[c % 2], pe_sem
                    )
                if it >= 2:
                    outh[it - 2].wait()
                inh[it] = pltpu.async_copy(
                    x_hbm.at[b, pl.ds(row0, _R), :], x_v.at[it % 2], in_sem
                )
            if it >= 1:
                itc = it - 1
                c, b = divmod(itc, B)
                row0 = base + c * _R
                inh[itc].wait()
                if b == 0:
                    peh[c].wait()
                xb = x_v.at[itc % 2]
                pb = pe_v.at[c % 2]

                def add_body(i, carry):
                    r = i >> 3
                    j = (i & 7) * 128
                    for u in range(8):
                        sl = pl.ds(j + u * 16, 16)
                        plsc.addupdate(xb.at[r, sl], pb[r, sl])
                    return carry

                lax.fori_loop(0, _R * 8, add_body, 0)
                outh[itc] = pltpu.async_copy(
                    xb, out_hbm.at[b, pl.ds(row0, _R), :], out_sem
                )
        outh[n_items - 2].wait()
        outh[n_items - 1].wait()

    return k(x, pe)


# speedup vs baseline: 4.0572x; 3.2294x over previous
import jax
import jax.numpy as jnp
from jax.experimental import pallas as pl
from jax.experimental.pallas import tpu as pltpu

_BS = 2048


def _add_body(x_ref, pe_ref, o_ref):
    o_ref[...] = x_ref[...] + pe_ref[...]


def kernel(x, pe):
    B, S, D = x.shape
    return pl.pallas_call(
        _add_body,
        grid=(S // _BS, B),
        in_specs=[
            pl.BlockSpec((1, _BS, D), lambda s, b: (b, s, 0)),
            pl.BlockSpec((_BS, D), lambda s, b: (s, 0)),
        ],
        out_specs=pl.BlockSpec((1, _BS, D), lambda s, b: (b, s, 0)),
        out_shape=jax.ShapeDtypeStruct((B, S, D), x.dtype),
        compiler_params=pltpu.CompilerParams(
            dimension_semantics=("arbitrary", "arbitrary"),
        ),
    )(x, pe)
